# baseline (device time: 1066675 ns/iter reference)
import jax
import jax.numpy as jnp
from jax import lax
from jax.experimental import pallas as pl
from jax.experimental.pallas import tpu as pltpu

N_CHUNKS = 8


def kernel(x):
    m_per, n = x.shape
    n_half = n // 2
    m_tot = 2 * m_per
    rows_c = m_per // N_CHUNKS

    def body(x_ref, out_ref, send_sems, recv_sems, copy_sem):
        my_x = lax.axis_index("x")
        my_y = lax.axis_index("y")
        peer_y = 1 - my_y

        barrier_sem = pltpu.get_barrier_semaphore()
        pl.semaphore_signal(
            barrier_sem,
            inc=1,
            device_id=(my_x, peer_y),
            device_id_type=pl.DeviceIdType.MESH,
        )
        pl.semaphore_wait(barrier_sem, 1)

        local = pltpu.make_async_copy(
            x_ref.at[:, pl.ds(my_y * n_half, n_half)],
            out_ref.at[pl.ds(my_y * m_per, m_per), :],
            copy_sem,
        )
        local.start()

        def rdma(k):
            return pltpu.make_async_remote_copy(
                src_ref=x_ref.at[
                    pl.ds(k * rows_c, rows_c), pl.ds(peer_y * n_half, n_half)
                ],
                dst_ref=out_ref.at[pl.ds(my_y * m_per + k * rows_c, rows_c), :],
                send_sem=send_sems.at[k],
                recv_sem=recv_sems.at[k],
                device_id=(my_x, peer_y),
                device_id_type=pl.DeviceIdType.MESH,
            )

        rdmas = [rdma(k) for k in range(N_CHUNKS)]
        for r in rdmas:
            r.start()
        for r in rdmas:
            r.wait()
        local.wait()

    return pl.pallas_call(
        body,
        out_shape=jax.ShapeDtypeStruct((m_tot, n_half), x.dtype),
        in_specs=[pl.BlockSpec(memory_space=pltpu.MemorySpace.HBM)],
        out_specs=pl.BlockSpec(memory_space=pltpu.MemorySpace.HBM),
        scratch_shapes=[
            pltpu.SemaphoreType.DMA((N_CHUNKS,)),
            pltpu.SemaphoreType.DMA((N_CHUNKS,)),
            pltpu.SemaphoreType.DMA,
        ],
        compiler_params=pltpu.CompilerParams(collective_id=0),
    )(x)


# device time: 1066666 ns/iter; 1.0000x vs baseline; 1.0000x over previous
import jax
import jax.numpy as jnp
from jax import lax
from jax.experimental import pallas as pl
from jax.experimental.pallas import tpu as pltpu

N_CHUNKS = 8


def kernel(x):
    m_per, n = x.shape
    n_half = n // 2
    m_tot = 2 * m_per
    rows_c = m_per // N_CHUNKS

    def body(x_ref, out_ref, stage, send_sems, recv_sems, stage_sems, copy_sem):
        my_x = lax.axis_index("x")
        my_y = lax.axis_index("y")
        peer_y = 1 - my_y

        barrier_sem = pltpu.get_barrier_semaphore()
        pl.semaphore_signal(
            barrier_sem,
            inc=1,
            device_id=(my_x, peer_y),
            device_id_type=pl.DeviceIdType.MESH,
        )
        pl.semaphore_wait(barrier_sem, 1)

        local = pltpu.make_async_copy(
            x_ref.at[:, pl.ds(my_y * n_half, n_half)],
            out_ref.at[pl.ds(my_y * m_per, m_per), :],
            copy_sem,
        )
        local.start()

        def stage_copy(k):
            return pltpu.make_async_copy(
                x_ref.at[pl.ds(k * rows_c, rows_c), pl.ds(peer_y * n_half, n_half)],
                stage.at[k % 2],
                stage_sems.at[k % 2],
            )

        def rdma(k):
            return pltpu.make_async_remote_copy(
                src_ref=stage.at[k % 2],
                dst_ref=out_ref.at[pl.ds(my_y * m_per + k * rows_c, rows_c), :],
                send_sem=send_sems.at[k],
                recv_sem=recv_sems.at[k],
                device_id=(my_x, peer_y),
                device_id_type=pl.DeviceIdType.MESH,
            )

        rdmas = [rdma(k) for k in range(N_CHUNKS)]
        for k in range(N_CHUNKS):
            c = stage_copy(k)
            c.start()
            c.wait()
            rdmas[k].start()
            rdmas[k].wait_send()
        for r in rdmas:
            r.wait_recv()
        local.wait()

    return pl.pallas_call(
        body,
        out_shape=jax.ShapeDtypeStruct((m_tot, n_half), x.dtype),
        in_specs=[pl.BlockSpec(memory_space=pltpu.MemorySpace.HBM)],
        out_specs=pl.BlockSpec(memory_space=pltpu.MemorySpace.HBM),
        scratch_shapes=[
            pltpu.VMEM((2, rows_c, n_half), x.dtype),
            pltpu.SemaphoreType.DMA((N_CHUNKS,)),
            pltpu.SemaphoreType.DMA((N_CHUNKS,)),
            pltpu.SemaphoreType.DMA((2,)),
            pltpu.SemaphoreType.DMA,
        ],
        compiler_params=pltpu.CompilerParams(collective_id=0),
    )(x)
